# baseline (speedup 1.0000x reference)
import functools

import jax
import jax.numpy as jnp
from jax import lax
from jax.experimental import pallas as pl
from jax.experimental.pallas import tpu as pltpu
from jax.experimental.pallas import tpu_sc as plsc

BATCH = 16384
EMBDIM = 64
CHUNK = 512
NFULL = 999936
NGROUPS = BATCH // 16

_info = plsc.get_sparse_core_info()
_NC, _NS = _info.num_cores, _info.num_subcores
_NW = _NC * _NS


def _make_gather():
    mesh = plsc.VectorSubcoreMesh(core_axis_name="c", subcore_axis_name="s")

    @functools.partial(
        pl.kernel,
        mesh=mesh,
        out_type=jax.ShapeDtypeStruct((BATCH + 16, 128), jnp.float32),
        scratch_types=[
            pltpu.VMEM((BATCH,), jnp.int32),
            pltpu.VMEM((BATCH,), jnp.int32),
            pltpu.VMEM((EMBDIM, CHUNK), jnp.float32),
            pltpu.VMEM((16, 128), jnp.float32),
            pltpu.SemaphoreType.DMA,
            pltpu.SemaphoreType.DMA,
        ],
        compiler_params=pltpu.CompilerParams(needs_layout_passes=False),
    )
    def gather_kernel(keys_hbm, pos_hbm, tableT_hbm, out_hbm,
                      keys_v, pos_v, chunk_v, ring_v, csem, osem):
        wid = lax.axis_index("s") * _NC + lax.axis_index("c")
        pltpu.sync_copy(keys_hbm, keys_v)
        pltpu.sync_copy(pos_hbm, pos_v)

        def srch(v):
            lo = jnp.int32(0)
            hi = jnp.int32(NGROUPS)
            for _ in range(10):
                mid = (lo + hi) // 2
                vec = keys_v[pl.ds(mid * 16, 16)]
                pred = vec[0] < v
                lo = jnp.where(pred, mid, lo)
                hi = jnp.where(pred, hi, mid)
            return lo

        nvalid = jnp.where(wid == 0, 62, 61)
        c0 = jnp.where(wid == 0, 0, 61 * wid + 1)

        iotas = [jnp.arange(16 * q, 16 * (q + 1), dtype=jnp.int32)
                 for q in range(EMBDIM // 16)]

        def do_chunk(k, carry):
            @pl.when(k < nvalid)
            def _():
                c = c0 + k
                col0 = pl.multiple_of(c * CHUNK, 128)
                cps = [
                    pltpu.async_copy(
                        tableT_hbm.at[pl.ds(8 * g8, 8), pl.ds(col0, CHUNK)],
                        chunk_v.at[pl.ds(8 * g8, 8)],
                        csem,
                    )
                    for g8 in range(EMBDIM // 8)
                ]
                for cp in cps:
                    cp.wait()
                g_lo = srch(c * CHUNK)
                g_hi = srch((c + 1) * CHUNK)

                def do_group(g, gcarry):
                    vec = keys_v[pl.ds(g * 16, 16)]
                    pvec = pos_v[pl.ds(g * 16, 16)]
                    mask = (vec >> 9) == c
                    pc = plsc.all_reduce_population_count(mask)

                    @pl.when(pc[0] > 0)
                    def _():
                        m32 = jnp.where(mask, 1, 0)
                        rlocs = jnp.where(mask, vec - c * CHUNK, 0)
                        selv = jnp.where(mask, pvec, BATCH)
                        for j in range(16):
                            @pl.when(m32[j] == 1)
                            def _():
                                idx1 = jnp.broadcast_to(rlocs[j], (16,))
                                for q in range(EMBDIM // 16):
                                    v = plsc.load_gather(
                                        chunk_v, [iotas[q], idx1])
                                    ring_v[j, pl.ds(16 * q, 16)] = v
                        pltpu.async_copy(
                            ring_v, out_hbm.at[selv], osem
                        ).wait()
                    return gcarry

                lax.fori_loop(g_lo, g_hi + 1, do_group, 0)
            return carry

        lax.fori_loop(0, 62, do_chunk, 0)

    return gather_kernel


_gather = _make_gather()


@jax.jit
def kernel(uid, table):
    uid = uid.astype(jnp.int32)
    keys, pos = lax.sort_key_val(uid, jnp.arange(BATCH, dtype=jnp.int32))
    main_out = _gather(keys, pos, table.T)[:BATCH, :EMBDIM]
    is_tail = uid >= NFULL
    onehot = (uid[:, None] == (NFULL + jnp.arange(EMBDIM))[None, :])
    tail_out = jnp.dot(
        onehot.astype(jnp.float32), table[NFULL:], precision="highest"
    )
    return jnp.where(is_tail[:, None], tail_out, main_out)


# speedup vs baseline: 4.1522x; 4.1522x over previous
import functools

import jax
import jax.numpy as jnp
from jax import lax
from jax.experimental import pallas as pl
from jax.experimental.pallas import tpu as pltpu
from jax.experimental.pallas import tpu_sc as plsc

BATCH = 16384
EMBDIM = 64

_info = plsc.get_sparse_core_info()
_NC, _NS = _info.num_cores, _info.num_subcores
_NW = _NC * _NS
_B_PER_W = BATCH // _NW


def _make_gather(D):
    mesh = plsc.VectorSubcoreMesh(core_axis_name="c", subcore_axis_name="s")

    @functools.partial(
        pl.kernel,
        mesh=mesh,
        out_type=jax.ShapeDtypeStruct((BATCH, D), jnp.float32),
        scratch_types=[
            pltpu.VMEM((_B_PER_W,), jnp.int32),
            pltpu.VMEM((_B_PER_W, D), jnp.float32),
            pltpu.SemaphoreType.DMA,
        ],
    )
    def gather_kernel(uid_hbm, table_hbm, out_hbm, idx_v, rows_v, sem):
        wid = lax.axis_index("s") * _NC + lax.axis_index("c")
        base = wid * _B_PER_W
        pltpu.sync_copy(uid_hbm.at[pl.ds(base, _B_PER_W)], idx_v)

        def enqueue(g, carry):
            vec = idx_v[pl.ds(g * 16, 16)]
            for j in range(16):
                r = vec[j]
                pltpu.async_copy(table_hbm.at[r], rows_v.at[g * 16 + j], sem)
            return carry

        lax.fori_loop(0, _B_PER_W // 16, enqueue, 0)

        def drain(i, carry):
            pltpu.make_async_copy(table_hbm.at[0], rows_v.at[0], sem).wait()
            return carry

        lax.fori_loop(0, _B_PER_W, drain, 0, unroll=8)
        pltpu.sync_copy(rows_v, out_hbm.at[pl.ds(base, _B_PER_W)])

    return gather_kernel


_gather = _make_gather(EMBDIM)


@jax.jit
def kernel(uid, table):
    return _gather(uid.astype(jnp.int32), table)
